# baseline (speedup 1.0000x reference)
import jax
import jax.numpy as jnp
from jax.experimental import pallas as pl
from jax.experimental.pallas import tpu as pltpu

N = 8192
D_MODEL = 1024
OUT_DIM = 256
E = 8
G = 64
BN = 512
NB = N // BN


def _moe_block_kernel(bidx_ref, didx_ref, emb_ref, W_ref, b_ref, out_ref):
    bidx = bidx_ref[0]
    g_iota = jax.lax.broadcasted_iota(jnp.int32, (BN, G), 1)
    onehot = bidx == g_iota
    didx = didx_ref[...]
    e_atom = jnp.sum(jnp.where(onehot, didx, 0), axis=1,
                     keepdims=True)
    x = emb_ref[...].astype(jnp.bfloat16)
    for e in range(E):
        mask = e_atom == e
        present = jnp.any(mask)

        @pl.when(present)
        def _(e=e, mask=mask):
            y = jnp.dot(x, W_ref[e], preferred_element_type=jnp.float32)
            y = y + b_ref[pl.ds(e, 1), :]
            out_ref[e] = jnp.where(mask, y, 0.0)

        @pl.when(jnp.logical_not(present))
        def _(e=e):
            out_ref[e] = jnp.zeros((BN, OUT_DIM), jnp.float32)


def kernel(emb, W, b, batch_idx, dataset_idx):
    bidx = batch_idx.astype(jnp.int32).reshape(NB, BN, 1)
    didx = dataset_idx.astype(jnp.int32).reshape(1, G)
    out = pl.pallas_call(
        _moe_block_kernel,
        grid=(NB,),
        in_specs=[
            pl.BlockSpec((1, BN, 1), lambda i: (i, 0, 0)),
            pl.BlockSpec((1, G), lambda i: (0, 0)),
            pl.BlockSpec((BN, D_MODEL), lambda i: (i, 0)),
            pl.BlockSpec((E, D_MODEL, OUT_DIM), lambda i: (0, 0, 0)),
            pl.BlockSpec((E, OUT_DIM), lambda i: (0, 0)),
        ],
        out_specs=pl.BlockSpec((E, BN, OUT_DIM), lambda i: (0, i, 0)),
        out_shape=jax.ShapeDtypeStruct((E, N, OUT_DIM), jnp.float32),
        compiler_params=pltpu.CompilerParams(
            dimension_semantics=("parallel",),
        ),
    )(bidx, didx, emb, W.astype(jnp.bfloat16), b)
    return out


# speedup vs baseline: 1.0774x; 1.0774x over previous
import jax
import jax.numpy as jnp
from jax.experimental import pallas as pl
from jax.experimental.pallas import tpu as pltpu

N = 8192
D_MODEL = 1024
OUT_DIM = 256
E = 8
G = 64
BN = 512
NB = N // BN


def _moe_block_kernel(bits_ref, bidx_ref, didx_ref, emb_ref, W_ref, b_ref,
                      out_ref):
    i = pl.program_id(0)
    bits = bits_ref[i]
    bidx = bidx_ref[0]
    g_iota = jax.lax.broadcasted_iota(jnp.int32, (BN, G), 1)
    onehot = bidx == g_iota
    didx = didx_ref[...]
    e_atom = jnp.sum(jnp.where(onehot, didx, 0), axis=1,
                     keepdims=True)
    x = emb_ref[...].astype(jnp.bfloat16)
    for e in range(E):
        present = ((bits >> e) & 1) == 1

        @pl.when(present)
        def _(e=e):
            mask = e_atom == e
            y = jnp.dot(x, W_ref[e].astype(jnp.bfloat16),
                        preferred_element_type=jnp.float32)
            y = y + b_ref[pl.ds(e, 1), :]
            out_ref[e] = jnp.where(mask, y, 0.0)

        @pl.when(jnp.logical_not(present))
        def _(e=e):
            out_ref[e] = jnp.zeros((BN, OUT_DIM), jnp.float32)


def kernel(emb, W, b, batch_idx, dataset_idx):
    bi = batch_idx.astype(jnp.int32)
    bidx = bi.reshape(NB, BN, 1)
    didx = dataset_idx.astype(jnp.int32).reshape(1, G)
    br = bi.reshape(NB, BN)
    g_lo = br[:, 0]
    g_hi = br[:, BN - 1]
    g_ar = jnp.arange(G, dtype=jnp.int32)
    rng = (g_ar[None, :] >= g_lo[:, None]) & (g_ar[None, :] <= g_hi[:, None])
    d32 = dataset_idx.astype(jnp.int32)
    presence = jnp.any(rng[:, :, None]
                       & (d32[None, :, None] == jnp.arange(E)[None, None, :]),
                       axis=1)
    bits = jnp.sum(presence.astype(jnp.int32)
                   << jnp.arange(E, dtype=jnp.int32)[None, :], axis=1)

    out = pl.pallas_call(
        _moe_block_kernel,
        grid_spec=pltpu.PrefetchScalarGridSpec(
            num_scalar_prefetch=1,
            grid=(NB,),
            in_specs=[
                pl.BlockSpec((1, BN, 1), lambda i, bits_ref: (i, 0, 0)),
                pl.BlockSpec((1, G), lambda i, bits_ref: (0, 0)),
                pl.BlockSpec((BN, D_MODEL), lambda i, bits_ref: (i, 0)),
                pl.BlockSpec((E, D_MODEL, OUT_DIM),
                             lambda i, bits_ref: (0, 0, 0)),
                pl.BlockSpec((E, OUT_DIM), lambda i, bits_ref: (0, 0)),
            ],
            out_specs=pl.BlockSpec((E, BN, OUT_DIM),
                                   lambda i, bits_ref: (0, i, 0)),
        ),
        out_shape=jax.ShapeDtypeStruct((E, N, OUT_DIM), jnp.float32),
        compiler_params=pltpu.CompilerParams(
            dimension_semantics=("parallel",),
        ),
    )(bits, bidx, didx, emb, W, b)
    return out
